# Initial kernel scaffold; baseline (speedup 1.0000x reference)
import jax
import jax.numpy as jnp
from jax.experimental import pallas as pl


def kernel(current_nodes, current_entities, current_masks, candidate_nodes, candidate_entities, candidate_relations, candidate_masks, node_embeddings, query_representations, entity_table, relation_table, W_nexthop, b_nexthop, W_cand, b_cand):
    raise NotImplementedError("write your pallas kernel here")


# speedup vs baseline: 9.5096x; 9.5096x over previous
import functools

import jax
import jax.numpy as jnp
from jax import lax
from jax.experimental import pallas as pl
from jax.experimental.pallas import tpu as pltpu
from jax.experimental.pallas import tpu_sc as plsc

B = 256
R = 16
MN = 64
H = 128
MAXN = 258
NROWS_NODE = B * MAXN
NCUR = B * R
NCAND = B * R * MN

NC = 2
NS = 16
NW = NC * NS


def _mm_body(x_ref, w_ref, o_ref):
    o_ref[...] = jnp.dot(x_ref[...], w_ref[...],
                         preferred_element_type=jnp.float32)


def _mm(x, w, bm):
    n = x.shape[0]
    grid = n // bm
    return pl.pallas_call(
        _mm_body,
        grid=(grid,),
        in_specs=[
            pl.BlockSpec((bm, x.shape[1]), lambda i: (i, 0)),
            pl.BlockSpec(w.shape, lambda i: (0, 0)),
        ],
        out_specs=pl.BlockSpec((bm, w.shape[1]), lambda i: (i, 0)),
        out_shape=jax.ShapeDtypeStruct((n, w.shape[1]), jnp.float32),
    )(x, w)


def _mm_bias_body(x_ref, w_ref, b_ref, o_ref):
    o_ref[...] = jnp.dot(x_ref[...], w_ref[...],
                         preferred_element_type=jnp.float32) + b_ref[...]


def _mm_bias(x, w, b):
    n = x.shape[0]
    return pl.pallas_call(
        _mm_bias_body,
        grid=(1,),
        in_specs=[
            pl.BlockSpec((n, x.shape[1]), lambda i: (0, 0)),
            pl.BlockSpec(w.shape, lambda i: (0, 0)),
            pl.BlockSpec((1, w.shape[1]), lambda i: (0, 0)),
        ],
        out_specs=pl.BlockSpec((n, w.shape[1]), lambda i: (0, 0)),
        out_shape=jax.ShapeDtypeStruct((n, w.shape[1]), jnp.float32),
    )(x, w, b)


def _state_body(a_ref, q_ref, e_ref, w_ref, b_ref, o_ref):
    x = jnp.concatenate([a_ref[...], q_ref[...], e_ref[...]], axis=-1)
    y = jnp.dot(x, w_ref[...], preferred_element_type=jnp.float32) + b_ref[...]
    y = jnp.maximum(y, 0.01 * y)
    o_ref[...] = y * (1.0 / jnp.sqrt(jnp.float32(H)))


def _state(cur_repr, qfull, cur_ent, w, b):
    n = cur_repr.shape[0]
    return pl.pallas_call(
        _state_body,
        grid=(1,),
        in_specs=[
            pl.BlockSpec((n, H), lambda i: (0, 0)),
            pl.BlockSpec((n, H), lambda i: (0, 0)),
            pl.BlockSpec((n, H), lambda i: (0, 0)),
            pl.BlockSpec(w.shape, lambda i: (0, 0)),
            pl.BlockSpec((1, H), lambda i: (0, 0)),
        ],
        out_specs=pl.BlockSpec((n, H), lambda i: (0, 0)),
        out_shape=jax.ShapeDtypeStruct((n, H), jnp.float32),
    )(cur_repr, qfull, cur_ent, w, b)


def _reduce_body(x_ref, s_ref, o_ref):
    o_ref[...] = jnp.dot(x_ref[...], s_ref[...],
                         preferred_element_type=jnp.float32)


def _reduce16(x, s, bm):
    n = x.shape[0]
    return pl.pallas_call(
        _reduce_body,
        grid=(n // bm,),
        in_specs=[
            pl.BlockSpec((bm, 128), lambda i: (i, 0)),
            pl.BlockSpec((128, 8), lambda i: (0, 0)),
        ],
        out_specs=pl.BlockSpec((bm, 8), lambda i: (i, 0)),
        out_shape=jax.ShapeDtypeStruct((n, 8), jnp.float32),
    )(x, s)


_CUR_PER_TILE = NCUR // NW


@functools.partial(
    pl.kernel,
    out_type=(
        jax.ShapeDtypeStruct((NCUR, H), jnp.float32),
        jax.ShapeDtypeStruct((NCUR, H), jnp.float32),
    ),
    mesh=plsc.VectorSubcoreMesh(core_axis_name="c", subcore_axis_name="s"),
    scratch_types=[
        pltpu.VMEM((_CUR_PER_TILE,), jnp.int32),
        pltpu.VMEM((_CUR_PER_TILE, H), jnp.float32),
        pltpu.VMEM((_CUR_PER_TILE,), jnp.int32),
        pltpu.VMEM((_CUR_PER_TILE, H), jnp.float32),
        pltpu.SemaphoreType.DMA,
    ],
)
def _cur_gather(nflat_hbm, etab_hbm, idxn_hbm, idxe_hbm,
                outn_hbm, oute_hbm,
                idxn_v, rown_v, idxe_v, rowe_v, sem):
    wid = lax.axis_index("s") * NC + lax.axis_index("c")
    base = wid * _CUR_PER_TILE
    pltpu.sync_copy(idxn_hbm.at[pl.ds(base, _CUR_PER_TILE)], idxn_v)
    pltpu.sync_copy(idxe_hbm.at[pl.ds(base, _CUR_PER_TILE)], idxe_v)
    cpn = pltpu.async_copy(nflat_hbm.at[idxn_v], rown_v, sem)
    cpe = pltpu.async_copy(etab_hbm.at[idxe_v], rowe_v, sem)
    cpn.wait()
    cpe.wait()
    pltpu.sync_copy(rown_v, outn_hbm.at[pl.ds(base, _CUR_PER_TILE)])
    pltpu.sync_copy(rowe_v, oute_hbm.at[pl.ds(base, _CUR_PER_TILE)])


_PER_TILE = NCAND // NW
_CHUNK = 128
_NCHUNK = _PER_TILE // _CHUNK
_SROWS = _PER_TILE // MN


@functools.partial(
    pl.kernel,
    out_type=jax.ShapeDtypeStruct((NCAND, 16), jnp.float32),
    mesh=plsc.VectorSubcoreMesh(core_axis_name="c", subcore_axis_name="s"),
    scratch_types=[
        pltpu.VMEM((_PER_TILE,), jnp.int32),
        pltpu.VMEM((_PER_TILE,), jnp.int32),
        pltpu.VMEM((_PER_TILE,), jnp.int32),
        pltpu.VMEM((_SROWS * H,), jnp.float32),
        pltpu.VMEM((_CHUNK, H), jnp.float32),
        pltpu.VMEM((_CHUNK, H), jnp.float32),
        pltpu.VMEM((_CHUNK, H), jnp.float32),
        pltpu.VMEM((_CHUNK, 16), jnp.float32),
        pltpu.SemaphoreType.DMA,
    ],
)
def _combine(tn_hbm, te_hbm, tr_hbm, idxn_hbm, idxe_hbm, idxr_hbm, state_hbm,
             out_hbm,
             idxn_v, idxe_v, idxr_v, state_v, bufn, bufe, bufr, accbuf, sem):
    wid = lax.axis_index("s") * NC + lax.axis_index("c")
    cbase = wid * _PER_TILE
    pbase = wid * _SROWS
    pltpu.sync_copy(idxn_hbm.at[pl.ds(cbase, _PER_TILE)], idxn_v)
    pltpu.sync_copy(idxe_hbm.at[pl.ds(cbase, _PER_TILE)], idxe_v)
    pltpu.sync_copy(idxr_hbm.at[pl.ds(cbase, _PER_TILE)], idxr_v)
    pltpu.sync_copy(state_hbm.at[pl.ds(pbase * H, _SROWS * H)], state_v)

    def chunk_body(c, carry):
        off = c * _CHUNK
        cpn = pltpu.async_copy(tn_hbm.at[idxn_v.at[pl.ds(off, _CHUNK)]],
                               bufn, sem)
        cpe = pltpu.async_copy(te_hbm.at[idxe_v.at[pl.ds(off, _CHUNK)]],
                               bufe, sem)
        cpr = pltpu.async_copy(tr_hbm.at[idxr_v.at[pl.ds(off, _CHUNK)]],
                               bufr, sem)
        cpn.wait()
        cpe.wait()
        cpr.wait()
        for half in range(_CHUNK // MN):
            srow = c * (_CHUNK // MN) + half
            svs = [state_v[pl.ds(srow * H + 16 * h, 16)]
                   for h in range(H // 16)]

            def cand_body(j, carry2, _half=half, _svs=svs):
                jj = _half * MN + j
                acc = jnp.zeros((16,), jnp.float32)
                for h in range(H // 16):
                    t = (bufn[jj, 16 * h:16 * (h + 1)]
                         + bufe[jj, 16 * h:16 * (h + 1)]
                         + bufr[jj, 16 * h:16 * (h + 1)])
                    t = jnp.maximum(t, 0.01 * t)
                    acc = acc + t * _svs[h]
                accbuf[jj, :] = acc
                return carry2

            lax.fori_loop(0, MN, cand_body, 0)
        pltpu.sync_copy(accbuf, out_hbm.at[pl.ds(cbase + off, _CHUNK)])
        return carry

    lax.fori_loop(0, _NCHUNK, chunk_body, 0)


def kernel(current_nodes, current_entities, current_masks,
           candidate_nodes, candidate_entities, candidate_relations,
           candidate_masks,
           node_embeddings, query_representations,
           entity_table, relation_table,
           W_nexthop, b_nexthop, W_cand, b_cand):
    f32 = jnp.float32
    i32 = jnp.int32
    nflat = node_embeddings.reshape(NROWS_NODE, H)

    boff_cur = (jnp.arange(B, dtype=i32) * MAXN)[:, None]
    boff_cand = boff_cur[:, :, None]
    idxn_cur = (current_nodes.astype(i32) + boff_cur).reshape(-1)
    idxe_cur = current_entities.astype(i32).reshape(-1)
    idxn = (candidate_nodes.astype(i32) + boff_cand).reshape(-1)
    idxe = candidate_entities.astype(i32).reshape(-1)
    idxr = candidate_relations.astype(i32).reshape(-1)

    tn = _mm(nflat, W_cand[:H], 512)
    te = _mm(entity_table, W_cand[H:2 * H], 2000)
    rel_pad = jnp.pad(relation_table, ((0, 512 - relation_table.shape[0]),
                                       (0, 0)))
    tr = _mm_bias(rel_pad, W_cand[2 * H:], b_cand.reshape(1, H))

    cur_repr, cur_ent = _cur_gather(nflat, entity_table, idxn_cur, idxe_cur)

    qfull = jnp.broadcast_to(query_representations[:, None, :],
                             (B, R, H)).reshape(NCUR, H)
    state = _state(cur_repr, qfull, cur_ent, W_nexthop,
                   b_nexthop.reshape(1, H))

    partials = _combine(tn, te, tr, idxn, idxe, idxr, state.reshape(-1))

    sel = (jnp.arange(128, dtype=i32)[:, None] // 16
           == jnp.arange(8, dtype=i32)[None, :]).astype(f32)
    scores = _reduce16(partials.reshape(NCAND // 8, 128), sel, 4096)
    scores = scores.reshape(B, R, MN)
    return jnp.where(candidate_masks, scores, f32(-100000.0))
